# baseline (speedup 1.0000x reference)
import functools

import jax
import jax.numpy as jnp
from jax import lax
from jax.experimental import pallas as pl
from jax.experimental.pallas import tpu as pltpu
from jax.experimental.pallas import tpu_sc as plsc

N = 10002
E = 320064
D = 128
N_PAD = 10240
NC = 2
NS = 16
NW = NC * NS
CHUNK = 128
G = 8
C = 80
NG = C // G
E_PAD = NW * C * CHUNK
ROWS_PER_TILE = N_PAD // NS


def _mesh():
    return plsc.VectorSubcoreMesh(
        core_axis_name="c", subcore_axis_name="s",
        num_cores=NC, num_subcores=NS)


@functools.lru_cache(maxsize=None)
def _make_segsum():
    def body(h_hbm, idx_hbm, zeros_hbm, out_hbm,
             idxg, rows, accum_sh, sem_a, sem_b):
        c = lax.axis_index("c")
        s = lax.axis_index("s")
        wid = c * NS + s
        r0 = s * ROWS_PER_TILE
        sems = (sem_a, sem_b)

        pltpu.sync_copy(zeros_hbm.at[pl.ds(r0, ROWS_PER_TILE)],
                        accum_sh.at[pl.ds(r0, ROWS_PER_TILE)])
        plsc.subcore_barrier()

        pltpu.sync_copy(idx_hbm.at[wid, pl.ds(0, G)], idxg.at[0])
        pltpu.async_copy(h_hbm.at[idxg.at[0, 0, 0]], rows.at[0], sem_a)

        @pl.loop(0, NG)
        def _group(g):
            gp = lax.rem(g, 2)

            @pl.when(g < NG - 1)
            def _():
                pltpu.sync_copy(idx_hbm.at[wid, pl.ds((g + 1) * G, G)],
                                idxg.at[1 - gp])

            for k in range(G):
                cur, nxt = k % 2, (k + 1) % 2
                if k < G - 1:
                    pltpu.async_copy(h_hbm.at[idxg.at[gp, k + 1, 0]],
                                     rows.at[nxt], sems[nxt])
                else:
                    @pl.when(g < NG - 1)
                    def _():
                        pltpu.async_copy(h_hbm.at[idxg.at[1 - gp, 0, 0]],
                                         rows.at[nxt], sems[nxt])
                pltpu.make_async_copy(h_hbm.at[pl.ds(0, CHUNK)],
                                      rows.at[cur], sems[cur]).wait()
                pltpu.sync_copy(rows.at[cur],
                                accum_sh.at[idxg.at[gp, k, 1]], add=True)

        plsc.subcore_barrier()
        pltpu.sync_copy(accum_sh.at[pl.ds(r0, ROWS_PER_TILE)],
                        out_hbm.at[c, pl.ds(r0, ROWS_PER_TILE)])

    return pl.kernel(
        body,
        out_type=(jax.ShapeDtypeStruct((NC, N_PAD, D), jnp.float32),),
        mesh=_mesh(),
        scratch_types=[
            pltpu.VMEM((2, G, 2, CHUNK), jnp.int32),
            pltpu.VMEM((2, CHUNK, D), jnp.float32),
            pltpu.VMEM_SHARED((N_PAD, D), jnp.float32),
            pltpu.SemaphoreType.DMA,
            pltpu.SemaphoreType.DMA,
        ],
        compiler_params=pltpu.CompilerParams(use_tc_tiling_on_sc=False))


@functools.lru_cache(maxsize=None)
def _make_deg():
    def body(idx_hbm, zeros16_hbm, ones_hbm, degout_hbm,
             idxg, ones_v, deg_sh):
        c = lax.axis_index("c")
        s = lax.axis_index("s")
        wid = c * NS + s
        r0 = s * ROWS_PER_TILE
        pltpu.sync_copy(zeros16_hbm.at[pl.ds(r0, ROWS_PER_TILE)],
                        deg_sh.at[pl.ds(r0, ROWS_PER_TILE)])
        pltpu.sync_copy(ones_hbm, ones_v)
        plsc.subcore_barrier()

        @pl.loop(0, NG)
        def _group(g):
            pltpu.sync_copy(idx_hbm.at[wid, pl.ds(g * G, G)], idxg)
            for k in range(G):
                pltpu.sync_copy(ones_v, deg_sh.at[idxg.at[k, 1]], add=True)

        plsc.subcore_barrier()
        pltpu.sync_copy(deg_sh.at[pl.ds(r0, ROWS_PER_TILE)],
                        degout_hbm.at[c, pl.ds(r0, ROWS_PER_TILE)])

    return pl.kernel(
        body,
        out_type=(jax.ShapeDtypeStruct((NC, N_PAD, 16), jnp.float32),),
        mesh=_mesh(),
        scratch_types=[
            pltpu.VMEM((G, 2, CHUNK), jnp.int32),
            pltpu.VMEM((CHUNK, 16), jnp.float32),
            pltpu.VMEM_SHARED((N_PAD, 16), jnp.float32),
        ],
        compiler_params=pltpu.CompilerParams(use_tc_tiling_on_sc=False))


_ROW_BLK = 512


def _layer_call(h, p0, p1, d0, d1, Ws, Wn, b, relu):
    def body(h_ref, p0_ref, p1_ref, d0_ref, d1_ref, ws_ref, wn_ref, b_ref,
             o_ref):
        deg = d0_ref[:, 0:1] + d1_ref[:, 0:1]
        rdeg = 1.0 / jnp.maximum(deg, 1.0)
        hn = (p0_ref[...] + p1_ref[...]) * rdeg
        acc = (jnp.dot(h_ref[...], ws_ref[...],
                       preferred_element_type=jnp.float32)
               + jnp.dot(hn, wn_ref[...], preferred_element_type=jnp.float32)
               + b_ref[...])
        o_ref[...] = jnp.maximum(acc, 0.0) if relu else acc

    grid = (N_PAD // _ROW_BLK,)
    row = lambda i: (i, 0)
    fixed = lambda i: (0, 0)
    return pl.pallas_call(
        body,
        grid=grid,
        in_specs=[
            pl.BlockSpec((_ROW_BLK, D), row),
            pl.BlockSpec((_ROW_BLK, D), row),
            pl.BlockSpec((_ROW_BLK, D), row),
            pl.BlockSpec((_ROW_BLK, 16), row),
            pl.BlockSpec((_ROW_BLK, 16), row),
            pl.BlockSpec((D, D), fixed),
            pl.BlockSpec((D, D), fixed),
            pl.BlockSpec((1, D), fixed),
        ],
        out_specs=pl.BlockSpec((_ROW_BLK, D), row),
        out_shape=jax.ShapeDtypeStruct((N_PAD, D), jnp.float32),
    )(h, p0, p1, d0, d1, Ws, Wn, b)


_R_PAD = 3336


def _predictor_call(sh, ph, nh, pw1, pb1, pw2, pb2, pw3, pb3):
    def body(s_ref, p_ref, n_ref, w1_ref, b1_ref, w2_ref, b2_ref, w3_ref,
             b3_ref, op_ref, on_ref):
        w1 = w1_ref[...]
        w2 = w2_ref[...]
        w3 = w3_ref[...]
        for z_in, o_ref in ((s_ref[...] * p_ref[...], op_ref),
                            (s_ref[...] * n_ref[...], on_ref)):
            z = jnp.maximum(
                jnp.dot(z_in, w1, preferred_element_type=jnp.float32)
                + b1_ref[...], 0.0)
            z = jnp.maximum(
                jnp.dot(z, w2, preferred_element_type=jnp.float32)
                + b2_ref[...], 0.0)
            o_ref[...] = (jnp.dot(z, w3, preferred_element_type=jnp.float32)
                          + b3_ref[...])

    return pl.pallas_call(
        body,
        out_shape=(jax.ShapeDtypeStruct((_R_PAD, 1), jnp.float32),
                   jax.ShapeDtypeStruct((_R_PAD, 1), jnp.float32)),
    )(sh, ph, nh, pw1, pb1.reshape(1, D), pw2, pb2.reshape(1, D), pw3,
      pb3.reshape(1, 1))


def kernel(x, edge_index, Ws0, Wn0, b0, Ws1, Wn1, b1, Ws2, Wn2, b2,
           pw1, pb1, pw2, pb2, pw3, pb3):
    src = edge_index[0]
    dst = edge_index[1]
    pad = E_PAD - E
    srcp = jnp.concatenate(
        [src, jnp.zeros((pad,), jnp.int32)]).reshape(NW, C, 1, CHUNK)
    dstp = jnp.concatenate(
        [dst, jnp.full((pad,), N_PAD - 1, jnp.int32)]).reshape(NW, C, 1, CHUNK)
    idxp = jnp.concatenate([srcp, dstp], axis=2)

    h = jnp.pad(x, ((0, N_PAD - N), (0, 0)))
    zeros = jnp.zeros((N_PAD, D), jnp.float32)
    zeros16 = jnp.zeros((N_PAD, 16), jnp.float32)
    ones = jnp.ones((CHUNK, 16), jnp.float32)

    (dp,) = _make_deg()(idxp, zeros16, ones)
    (p,) = _make_segsum()(h, idxp, zeros)
    h = _layer_call(h, p[0], p[1], dp[0], dp[1], Ws0, Wn0,
                    b0.reshape(1, D), True)
    (p,) = _make_segsum()(h, idxp, zeros)
    h = _layer_call(h, p[0], p[1], dp[0], dp[1], Ws1, Wn1,
                    b1.reshape(1, D), True)
    (p,) = _make_segsum()(h, idxp, zeros)
    h = _layer_call(h, p[0], p[1], dp[0], dp[1], Ws2, Wn2,
                    b2.reshape(1, D), False)

    third = N // 3
    sh = jnp.pad(h[0:third], ((0, _R_PAD - third), (0, 0)))
    ph = jnp.pad(h[third:2 * third], ((0, _R_PAD - third), (0, 0)))
    nh = jnp.pad(h[2 * third:N], ((0, _R_PAD - third), (0, 0)))
    h_pos, h_neg = _predictor_call(sh, ph, nh, pw1, pb1, pw2, pb2, pw3, pb3)
    return (h_pos[:third], h_neg[:third])


# speedup vs baseline: 1.0060x; 1.0060x over previous
import functools

import jax
import jax.numpy as jnp
from jax import lax
from jax.experimental import pallas as pl
from jax.experimental.pallas import tpu as pltpu
from jax.experimental.pallas import tpu_sc as plsc

N = 10002
E = 320064
D = 128
N_PAD = 10240
NC = 2
NS = 16
NW = NC * NS
CHUNK = 128
G = 8
C = 80
NG = C // G
E_PAD = NW * C * CHUNK
ROWS_PER_TILE = N_PAD // NS


def _mesh():
    return plsc.VectorSubcoreMesh(
        core_axis_name="c", subcore_axis_name="s",
        num_cores=NC, num_subcores=NS)


@functools.lru_cache(maxsize=None)
def _make_segsum():
    def body(h_hbm, idx_hbm, zeros_hbm, out_hbm,
             idxg, rows, accum_sh, sem_a, sem_b):
        c = lax.axis_index("c")
        s = lax.axis_index("s")
        wid = c * NS + s
        r0 = s * ROWS_PER_TILE
        sems = (sem_a, sem_b)

        pltpu.sync_copy(zeros_hbm.at[pl.ds(r0, ROWS_PER_TILE)],
                        accum_sh.at[pl.ds(r0, ROWS_PER_TILE)])
        plsc.subcore_barrier()

        pltpu.sync_copy(idx_hbm.at[wid, pl.ds(0, G)], idxg.at[0])
        pltpu.async_copy(h_hbm.at[idxg.at[0, 0, 0]], rows.at[0], sem_a)

        @pl.loop(0, NG)
        def _group(g):
            gp = lax.rem(g, 2)

            @pl.when(g < NG - 1)
            def _():
                pltpu.sync_copy(idx_hbm.at[wid, pl.ds((g + 1) * G, G)],
                                idxg.at[1 - gp])

            for k in range(G):
                cur, nxt = k % 2, (k + 1) % 2
                if k < G - 1:
                    pltpu.async_copy(h_hbm.at[idxg.at[gp, k + 1, 0]],
                                     rows.at[nxt], sems[nxt])
                else:
                    @pl.when(g < NG - 1)
                    def _():
                        pltpu.async_copy(h_hbm.at[idxg.at[1 - gp, 0, 0]],
                                         rows.at[nxt], sems[nxt])
                pltpu.make_async_copy(h_hbm.at[pl.ds(0, CHUNK)],
                                      rows.at[cur], sems[cur]).wait()

        plsc.subcore_barrier()
        pltpu.sync_copy(accum_sh.at[pl.ds(r0, ROWS_PER_TILE)],
                        out_hbm.at[c, pl.ds(r0, ROWS_PER_TILE)])

    return pl.kernel(
        body,
        out_type=(jax.ShapeDtypeStruct((NC, N_PAD, D), jnp.float32),),
        mesh=_mesh(),
        scratch_types=[
            pltpu.VMEM((2, G, 2, CHUNK), jnp.int32),
            pltpu.VMEM((2, CHUNK, D), jnp.float32),
            pltpu.VMEM_SHARED((N_PAD, D), jnp.float32),
            pltpu.SemaphoreType.DMA,
            pltpu.SemaphoreType.DMA,
        ],
        compiler_params=pltpu.CompilerParams(use_tc_tiling_on_sc=False))


@functools.lru_cache(maxsize=None)
def _make_deg():
    def body(idx_hbm, zeros16_hbm, ones_hbm, degout_hbm,
             idxg, ones_v, deg_sh):
        c = lax.axis_index("c")
        s = lax.axis_index("s")
        wid = c * NS + s
        r0 = s * ROWS_PER_TILE
        pltpu.sync_copy(zeros16_hbm.at[pl.ds(r0, ROWS_PER_TILE)],
                        deg_sh.at[pl.ds(r0, ROWS_PER_TILE)])
        pltpu.sync_copy(ones_hbm, ones_v)
        plsc.subcore_barrier()

        @pl.loop(0, NG)
        def _group(g):
            pltpu.sync_copy(idx_hbm.at[wid, pl.ds(g * G, G)], idxg)
            for k in range(G):
                pltpu.sync_copy(ones_v, deg_sh.at[idxg.at[k, 1]], add=True)

        plsc.subcore_barrier()
        pltpu.sync_copy(deg_sh.at[pl.ds(r0, ROWS_PER_TILE)],
                        degout_hbm.at[c, pl.ds(r0, ROWS_PER_TILE)])

    return pl.kernel(
        body,
        out_type=(jax.ShapeDtypeStruct((NC, N_PAD, 16), jnp.float32),),
        mesh=_mesh(),
        scratch_types=[
            pltpu.VMEM((G, 2, CHUNK), jnp.int32),
            pltpu.VMEM((CHUNK, 16), jnp.float32),
            pltpu.VMEM_SHARED((N_PAD, 16), jnp.float32),
        ],
        compiler_params=pltpu.CompilerParams(use_tc_tiling_on_sc=False))


_ROW_BLK = 512


def _layer_call(h, p0, p1, d0, d1, Ws, Wn, b, relu):
    def body(h_ref, p0_ref, p1_ref, d0_ref, d1_ref, ws_ref, wn_ref, b_ref,
             o_ref):
        deg = d0_ref[:, 0:1] + d1_ref[:, 0:1]
        rdeg = 1.0 / jnp.maximum(deg, 1.0)
        hn = (p0_ref[...] + p1_ref[...]) * rdeg
        acc = (jnp.dot(h_ref[...], ws_ref[...],
                       preferred_element_type=jnp.float32)
               + jnp.dot(hn, wn_ref[...], preferred_element_type=jnp.float32)
               + b_ref[...])
        o_ref[...] = jnp.maximum(acc, 0.0) if relu else acc

    grid = (N_PAD // _ROW_BLK,)
    row = lambda i: (i, 0)
    fixed = lambda i: (0, 0)
    return pl.pallas_call(
        body,
        grid=grid,
        in_specs=[
            pl.BlockSpec((_ROW_BLK, D), row),
            pl.BlockSpec((_ROW_BLK, D), row),
            pl.BlockSpec((_ROW_BLK, D), row),
            pl.BlockSpec((_ROW_BLK, 16), row),
            pl.BlockSpec((_ROW_BLK, 16), row),
            pl.BlockSpec((D, D), fixed),
            pl.BlockSpec((D, D), fixed),
            pl.BlockSpec((1, D), fixed),
        ],
        out_specs=pl.BlockSpec((_ROW_BLK, D), row),
        out_shape=jax.ShapeDtypeStruct((N_PAD, D), jnp.float32),
    )(h, p0, p1, d0, d1, Ws, Wn, b)


_R_PAD = 3336


def _predictor_call(sh, ph, nh, pw1, pb1, pw2, pb2, pw3, pb3):
    def body(s_ref, p_ref, n_ref, w1_ref, b1_ref, w2_ref, b2_ref, w3_ref,
             b3_ref, op_ref, on_ref):
        w1 = w1_ref[...]
        w2 = w2_ref[...]
        w3 = w3_ref[...]
        for z_in, o_ref in ((s_ref[...] * p_ref[...], op_ref),
                            (s_ref[...] * n_ref[...], on_ref)):
            z = jnp.maximum(
                jnp.dot(z_in, w1, preferred_element_type=jnp.float32)
                + b1_ref[...], 0.0)
            z = jnp.maximum(
                jnp.dot(z, w2, preferred_element_type=jnp.float32)
                + b2_ref[...], 0.0)
            o_ref[...] = (jnp.dot(z, w3, preferred_element_type=jnp.float32)
                          + b3_ref[...])

    return pl.pallas_call(
        body,
        out_shape=(jax.ShapeDtypeStruct((_R_PAD, 1), jnp.float32),
                   jax.ShapeDtypeStruct((_R_PAD, 1), jnp.float32)),
    )(sh, ph, nh, pw1, pb1.reshape(1, D), pw2, pb2.reshape(1, D), pw3,
      pb3.reshape(1, 1))


def kernel(x, edge_index, Ws0, Wn0, b0, Ws1, Wn1, b1, Ws2, Wn2, b2,
           pw1, pb1, pw2, pb2, pw3, pb3):
    src = edge_index[0]
    dst = edge_index[1]
    pad = E_PAD - E
    srcp = jnp.concatenate(
        [src, jnp.zeros((pad,), jnp.int32)]).reshape(NW, C, 1, CHUNK)
    dstp = jnp.concatenate(
        [dst, jnp.full((pad,), N_PAD - 1, jnp.int32)]).reshape(NW, C, 1, CHUNK)
    idxp = jnp.concatenate([srcp, dstp], axis=2)

    h = jnp.pad(x, ((0, N_PAD - N), (0, 0)))
    zeros = jnp.zeros((N_PAD, D), jnp.float32)
    zeros16 = jnp.zeros((N_PAD, 16), jnp.float32)
    ones = jnp.ones((CHUNK, 16), jnp.float32)

    (dp,) = _make_deg()(idxp, zeros16, ones)
    (p,) = _make_segsum()(h, idxp, zeros)
    h = _layer_call(h, p[0], p[1], dp[0], dp[1], Ws0, Wn0,
                    b0.reshape(1, D), True)
    (p,) = _make_segsum()(h, idxp, zeros)
    h = _layer_call(h, p[0], p[1], dp[0], dp[1], Ws1, Wn1,
                    b1.reshape(1, D), True)
    (p,) = _make_segsum()(h, idxp, zeros)
    h = _layer_call(h, p[0], p[1], dp[0], dp[1], Ws2, Wn2,
                    b2.reshape(1, D), False)

    third = N // 3
    sh = jnp.pad(h[0:third], ((0, _R_PAD - third), (0, 0)))
    ph = jnp.pad(h[third:2 * third], ((0, _R_PAD - third), (0, 0)))
    nh = jnp.pad(h[2 * third:N], ((0, _R_PAD - third), (0, 0)))
    h_pos, h_neg = _predictor_call(sh, ph, nh, pw1, pb1, pw2, pb2, pw3, pb3)
    return (h_pos[:third], h_neg[:third])


# speedup vs baseline: 2.0532x; 2.0410x over previous
import functools

import jax
import jax.numpy as jnp
from jax import lax
from jax.experimental import pallas as pl
from jax.experimental.pallas import tpu as pltpu
from jax.experimental.pallas import tpu_sc as plsc

N = 10002
E = 320064
D = 128
N_PAD = 10240
NC = 2
NS = 16
NW = NC * NS
CHUNK = 128
G = 8
C = 80
NG = C // G
E_PAD = NW * C * CHUNK
ROWS_PER_TILE = N_PAD // NS


def _mesh():
    return plsc.VectorSubcoreMesh(
        core_axis_name="c", subcore_axis_name="s",
        num_cores=NC, num_subcores=NS)


DH = D // NC
C2 = C * NC
NG2 = C2 // G


@functools.lru_cache(maxsize=None)
def _make_segsum():
    def body(hs_hbm, idx_hbm, zeros_hbm, out_hbm,
             idxg, rows, h_sh, accum_sh, sem_a, sem_b):
        c = lax.axis_index("c")
        s = lax.axis_index("s")
        r0 = s * ROWS_PER_TILE
        sems = (sem_a, sem_b)

        pltpu.sync_copy(hs_hbm.at[c, pl.ds(r0, ROWS_PER_TILE)],
                        h_sh.at[pl.ds(r0, ROWS_PER_TILE)])
        pltpu.sync_copy(zeros_hbm.at[pl.ds(r0, ROWS_PER_TILE)],
                        accum_sh.at[pl.ds(r0, ROWS_PER_TILE)])
        plsc.subcore_barrier()

        pltpu.sync_copy(idx_hbm.at[s, pl.ds(0, G)], idxg.at[0])
        pltpu.async_copy(h_sh.at[idxg.at[0, 0, 0]], rows.at[0], sem_a)

        @pl.loop(0, NG2)
        def _group(g):
            gp = lax.rem(g, 2)

            @pl.when(g < NG2 - 1)
            def _():
                pltpu.sync_copy(idx_hbm.at[s, pl.ds((g + 1) * G, G)],
                                idxg.at[1 - gp])

            for k in range(G):
                cur, nxt = k % 2, (k + 1) % 2
                if k < G - 1:
                    pltpu.async_copy(h_sh.at[idxg.at[gp, k + 1, 0]],
                                     rows.at[nxt], sems[nxt])
                else:
                    @pl.when(g < NG2 - 1)
                    def _():
                        pltpu.async_copy(h_sh.at[idxg.at[1 - gp, 0, 0]],
                                         rows.at[nxt], sems[nxt])
                pltpu.make_async_copy(h_sh.at[pl.ds(0, CHUNK)],
                                      rows.at[cur], sems[cur]).wait()
                pltpu.sync_copy(rows.at[cur],
                                accum_sh.at[idxg.at[gp, k, 1]], add=True)

        plsc.subcore_barrier()
        pltpu.sync_copy(accum_sh.at[pl.ds(r0, ROWS_PER_TILE)],
                        out_hbm.at[c, pl.ds(r0, ROWS_PER_TILE)])

    return pl.kernel(
        body,
        out_type=(jax.ShapeDtypeStruct((NC, N_PAD, DH), jnp.float32),),
        mesh=_mesh(),
        scratch_types=[
            pltpu.VMEM((2, G, 2, CHUNK), jnp.int32),
            pltpu.VMEM((2, CHUNK, DH), jnp.float32),
            pltpu.VMEM_SHARED((N_PAD, DH), jnp.float32),
            pltpu.VMEM_SHARED((N_PAD, DH), jnp.float32),
            pltpu.SemaphoreType.DMA,
            pltpu.SemaphoreType.DMA,
        ],
        compiler_params=pltpu.CompilerParams(use_tc_tiling_on_sc=False))


@functools.lru_cache(maxsize=None)
def _make_deg():
    def body(idx_hbm, zeros16_hbm, ones_hbm, degout_hbm,
             idxg, ones_v, deg_sh):
        c = lax.axis_index("c")
        s = lax.axis_index("s")
        wid = c * NS + s
        r0 = s * ROWS_PER_TILE
        pltpu.sync_copy(zeros16_hbm.at[pl.ds(r0, ROWS_PER_TILE)],
                        deg_sh.at[pl.ds(r0, ROWS_PER_TILE)])
        pltpu.sync_copy(ones_hbm, ones_v)
        plsc.subcore_barrier()

        @pl.loop(0, NG)
        def _group(g):
            pltpu.sync_copy(idx_hbm.at[wid, pl.ds(g * G, G)], idxg)
            for k in range(G):
                pltpu.sync_copy(ones_v, deg_sh.at[idxg.at[k, 1]], add=True)

        plsc.subcore_barrier()
        pltpu.sync_copy(deg_sh.at[pl.ds(r0, ROWS_PER_TILE)],
                        degout_hbm.at[c, pl.ds(r0, ROWS_PER_TILE)])

    return pl.kernel(
        body,
        out_type=(jax.ShapeDtypeStruct((NC, N_PAD, 16), jnp.float32),),
        mesh=_mesh(),
        scratch_types=[
            pltpu.VMEM((G, 2, CHUNK), jnp.int32),
            pltpu.VMEM((CHUNK, 16), jnp.float32),
            pltpu.VMEM_SHARED((N_PAD, 16), jnp.float32),
        ],
        compiler_params=pltpu.CompilerParams(use_tc_tiling_on_sc=False))


_ROW_BLK = 512


def _layer_call(h, p0, p1, d0, d1, Ws, Wn, b, relu):
    def body(h_ref, p0_ref, p1_ref, d0_ref, d1_ref, ws_ref, wn_ref, b_ref,
             o_ref):
        deg = d0_ref[:, 0:1] + d1_ref[:, 0:1]
        rdeg = 1.0 / jnp.maximum(deg, 1.0)
        hn = jnp.concatenate([p0_ref[...], p1_ref[...]], axis=1) * rdeg
        acc = (jnp.dot(h_ref[...], ws_ref[...],
                       preferred_element_type=jnp.float32)
               + jnp.dot(hn, wn_ref[...], preferred_element_type=jnp.float32)
               + b_ref[...])
        o_ref[...] = jnp.maximum(acc, 0.0) if relu else acc

    grid = (N_PAD // _ROW_BLK,)
    row = lambda i: (i, 0)
    fixed = lambda i: (0, 0)
    return pl.pallas_call(
        body,
        grid=grid,
        in_specs=[
            pl.BlockSpec((_ROW_BLK, D), row),
            pl.BlockSpec((_ROW_BLK, DH), row),
            pl.BlockSpec((_ROW_BLK, DH), row),
            pl.BlockSpec((_ROW_BLK, 16), row),
            pl.BlockSpec((_ROW_BLK, 16), row),
            pl.BlockSpec((D, D), fixed),
            pl.BlockSpec((D, D), fixed),
            pl.BlockSpec((1, D), fixed),
        ],
        out_specs=pl.BlockSpec((_ROW_BLK, D), row),
        out_shape=jax.ShapeDtypeStruct((N_PAD, D), jnp.float32),
    )(h, p0, p1, d0, d1, Ws, Wn, b)


_R_PAD = 3336


def _predictor_call(sh, ph, nh, pw1, pb1, pw2, pb2, pw3, pb3):
    def body(s_ref, p_ref, n_ref, w1_ref, b1_ref, w2_ref, b2_ref, w3_ref,
             b3_ref, op_ref, on_ref):
        w1 = w1_ref[...]
        w2 = w2_ref[...]
        w3 = w3_ref[...]
        for z_in, o_ref in ((s_ref[...] * p_ref[...], op_ref),
                            (s_ref[...] * n_ref[...], on_ref)):
            z = jnp.maximum(
                jnp.dot(z_in, w1, preferred_element_type=jnp.float32)
                + b1_ref[...], 0.0)
            z = jnp.maximum(
                jnp.dot(z, w2, preferred_element_type=jnp.float32)
                + b2_ref[...], 0.0)
            o_ref[...] = (jnp.dot(z, w3, preferred_element_type=jnp.float32)
                          + b3_ref[...])

    return pl.pallas_call(
        body,
        out_shape=(jax.ShapeDtypeStruct((_R_PAD, 1), jnp.float32),
                   jax.ShapeDtypeStruct((_R_PAD, 1), jnp.float32)),
    )(sh, ph, nh, pw1, pb1.reshape(1, D), pw2, pb2.reshape(1, D), pw3,
      pb3.reshape(1, 1))


def kernel(x, edge_index, Ws0, Wn0, b0, Ws1, Wn1, b1, Ws2, Wn2, b2,
           pw1, pb1, pw2, pb2, pw3, pb3):
    src = edge_index[0]
    dst = edge_index[1]
    pad = E_PAD - E
    srcp = jnp.concatenate(
        [src, jnp.zeros((pad,), jnp.int32)]).reshape(NW, C, 1, CHUNK)
    dstp = jnp.concatenate(
        [dst, jnp.full((pad,), N_PAD - 1, jnp.int32)]).reshape(NW, C, 1, CHUNK)
    idxp = jnp.concatenate([srcp, dstp], axis=2)
    idxp2 = idxp.reshape(NS, C2, 2, CHUNK)

    h = jnp.pad(x, ((0, N_PAD - N), (0, 0)))
    zeros64 = jnp.zeros((N_PAD, DH), jnp.float32)
    zeros16 = jnp.zeros((N_PAD, 16), jnp.float32)
    ones = jnp.ones((CHUNK, 16), jnp.float32)

    (dp,) = _make_deg()(idxp, zeros16, ones)
    for Ws, Wn, b, relu in ((Ws0, Wn0, b0, True), (Ws1, Wn1, b1, True),
                            (Ws2, Wn2, b2, False)):
        hs = jnp.stack([h[:, :DH], h[:, DH:]])
        (p,) = _make_segsum()(hs, idxp2, zeros64)
        h = _layer_call(h, p[0], p[1], dp[0], dp[1], Ws, Wn,
                        b.reshape(1, D), relu)

    third = N // 3
    sh = jnp.pad(h[0:third], ((0, _R_PAD - third), (0, 0)))
    ph = jnp.pad(h[third:2 * third], ((0, _R_PAD - third), (0, 0)))
    nh = jnp.pad(h[2 * third:N], ((0, _R_PAD - third), (0, 0)))
    h_pos, h_neg = _predictor_call(sh, ph, nh, pw1, pb1, pw2, pb2, pw3, pb3)
    return (h_pos[:third], h_neg[:third])
